# baseline (speedup 1.0000x reference)
import jax
import jax.numpy as jnp
from jax.experimental import pallas as pl
from jax.experimental.pallas import tpu as pltpu


def _fused_kernel(lbl_ref, x_ref, w1_ref, w2_ref, feat_ref, lab_ref):
    nj = x_ref.shape[2]
    for j in range(nj):
        xj = x_ref[0, :, j, :].astype(jnp.bfloat16)
        hid = jnp.dot(w1_ref[...], xj, preferred_element_type=jnp.float32)
        hb = jnp.maximum(hid, 0.0).astype(jnp.bfloat16)
        feat_ref[0, :, j, :] = jnp.dot(
            w2_ref[...], hb, preferred_element_type=jnp.float32)
    b = pl.program_id(0)
    lab_ref[...] = jnp.full(lab_ref.shape, lbl_ref[b], dtype=jnp.int32)


def kernel(x, w1, w2, need_backprop):
    B, C, H, W = x.shape
    hidden = w1.shape[0]
    out_c = w2.shape[0]
    HW = H * W
    J = HW // 128

    x_v = x.reshape(B, C, J, 128)

    lbl = need_backprop.astype(jnp.float32).astype(jnp.int32)
    w1b = w1.astype(jnp.bfloat16)
    w2b = w2.astype(jnp.bfloat16)

    feat, label = pl.pallas_call(
        _fused_kernel,
        out_shape=(
            jax.ShapeDtypeStruct((B, out_c, J, 128), x.dtype),
            jax.ShapeDtypeStruct((B, J, 128), jnp.int32),
        ),
        grid_spec=pltpu.PrefetchScalarGridSpec(
            num_scalar_prefetch=1,
            grid=(B,),
            in_specs=[
                pl.BlockSpec((1, C, J, 128), lambda b, lbl: (b, 0, 0, 0)),
                pl.BlockSpec((hidden, C), lambda b, lbl: (0, 0)),
                pl.BlockSpec((out_c, hidden), lambda b, lbl: (0, 0)),
            ],
            out_specs=(
                pl.BlockSpec((1, out_c, J, 128), lambda b, lbl: (b, 0, 0, 0)),
                pl.BlockSpec((1, J, 128), lambda b, lbl: (b, 0, 0)),
            ),
        ),
        compiler_params=pltpu.CompilerParams(
            dimension_semantics=("parallel",)),
    )(lbl, x_v, w1b, w2b)

    return feat.reshape(B, out_c, H, W), label.reshape(B, H, W)


# speedup vs baseline: 1.9156x; 1.9156x over previous
import jax
import jax.numpy as jnp
from jax.experimental import pallas as pl
from jax.experimental.pallas import tpu as pltpu


def _fused_kernel(lbl_ref, x_any, w1_ref, w2_ref, feat_ref, lab_ref,
                  xbuf, sems):
    b = pl.program_id(0)
    nb = pl.num_programs(0)
    slot = jax.lax.rem(b, 2)

    @pl.when(b == 0)
    def _start_first():
        pltpu.make_async_copy(x_any.at[0], xbuf.at[0], sems.at[0]).start()

    @pl.when(b + 1 < nb)
    def _prefetch_next():
        nxt = jax.lax.rem(b + 1, 2)
        pltpu.make_async_copy(x_any.at[b + 1], xbuf.at[nxt], sems.at[nxt]).start()

    pltpu.make_async_copy(xbuf.at[slot], xbuf.at[slot], sems.at[slot]).wait()

    xb = xbuf[slot].astype(jnp.bfloat16)
    hid = jnp.dot(w1_ref[...], xb, preferred_element_type=jnp.float32)
    hb = jnp.maximum(hid, 0.0).astype(jnp.bfloat16)
    feat_ref[0] = jnp.dot(w2_ref[...], hb, preferred_element_type=jnp.float32)
    lab_ref[...] = jnp.full(lab_ref.shape, lbl_ref[b], dtype=jnp.int32)


def kernel(x, w1, w2, need_backprop):
    B, C, H, W = x.shape
    hidden = w1.shape[0]
    out_c = w2.shape[0]
    HW = H * W

    x_r = x.reshape(B, C, HW)

    lbl = need_backprop.astype(jnp.float32).astype(jnp.int32)
    w1b = w1.astype(jnp.bfloat16)
    w2b = w2.astype(jnp.bfloat16)

    feat, lab = pl.pallas_call(
        _fused_kernel,
        out_shape=(
            jax.ShapeDtypeStruct((B, out_c, HW), x.dtype),
            jax.ShapeDtypeStruct((B, 1, HW), jnp.int32),
        ),
        grid_spec=pltpu.PrefetchScalarGridSpec(
            num_scalar_prefetch=1,
            grid=(B,),
            in_specs=[
                pl.BlockSpec(memory_space=pl.ANY),
                pl.BlockSpec((hidden, C), lambda b, lbl: (0, 0)),
                pl.BlockSpec((out_c, hidden), lambda b, lbl: (0, 0)),
            ],
            out_specs=(
                pl.BlockSpec((1, out_c, HW), lambda b, lbl: (b, 0, 0)),
                pl.BlockSpec((1, 1, HW), lambda b, lbl: (b, 0, 0)),
            ),
            scratch_shapes=[
                pltpu.VMEM((2, C, HW), jnp.float32),
                pltpu.SemaphoreType.DMA((2,)),
            ],
        ),
        compiler_params=pltpu.CompilerParams(
            dimension_semantics=("arbitrary",)),
    )(lbl, x_r, w1b, w2b)

    return feat.reshape(B, out_c, H, W), lab.reshape(B, H, W)
